# baseline (speedup 1.0000x reference)
import functools
import math

import ml_dtypes
import numpy as np
import jax
import jax.numpy as jnp
from jax import lax
from jax.experimental import pallas as pl
from jax.experimental.pallas import tpu as pltpu
from jax.experimental.pallas import tpu_sc as plsc

_D = 768
_B = 4
_S = 2048
_NW = 32
_S_PER_W = _S // _NW
_C = 64
_NCH = _B * (_S_PER_W // _C)
_NBUF = 2
_BLK = _D // 32


def _positional_encoding_np(max_len, d_model):
    position = np.arange(max_len, dtype=np.float32)[:, None]
    div_term = np.exp(
        np.arange(0, d_model, 2, dtype=np.float32) * -(math.log(10000.0) / d_model)
    )
    pe = np.zeros((max_len, d_model), dtype=np.float32)
    pe[:, 0::2] = np.sin(position * div_term)
    pe[:, 1::2] = np.cos(position * div_term)
    return pe


_PE = _positional_encoding_np(_S, _D)
_PE_IL = _PE.reshape(_S, _BLK, 2, 16).transpose(0, 1, 3, 2).reshape(_S, _D)
_PE_U32 = (
    _PE_IL.astype(ml_dtypes.bfloat16).view(np.uint32).reshape(_S, _D // 2)
)

_mesh = plsc.VectorSubcoreMesh(core_axis_name="c", subcore_axis_name="s")


@functools.partial(
    pl.kernel,
    mesh=_mesh,
    out_type=jax.ShapeDtypeStruct((_B * _S, _D), jnp.float32),
    scratch_types=[
        pltpu.VMEM((_B, _S_PER_W), jnp.int32),
        pltpu.VMEM((_NBUF, _C, _D), jnp.float32),
        pltpu.VMEM((_S_PER_W, _D // 2), jnp.uint32),
        pltpu.SemaphoreType.DMA((_NBUF,)),
        pltpu.SemaphoreType.DMA,
        pltpu.SemaphoreType.DMA,
        pltpu.SemaphoreType.DMA((2 * _NBUF,)),
    ],
)
def _embed_pe(table, pe, idx, out, idx_v, rows_v, pe_v, gsem, psem, isem, osem):
    wid = lax.axis_index("s") * 2 + lax.axis_index("c")
    s_base = wid * _S_PER_W

    pe_desc = pltpu.async_copy(pe.at[pl.ds(s_base, _S_PER_W)], pe_v, psem)
    idx_descs = [
        pltpu.async_copy(idx.at[b, pl.ds(s_base, _S_PER_W)], idx_v.at[b], isem)
        for b in range(_B)
    ]

    def start_gather(j):
        idx_descs[j].wait()
        return pltpu.async_copy(
            table.at[idx_v.at[j]], rows_v.at[j % _NBUF], gsem.at[j % _NBUF]
        )

    _H = _C // 2

    def add_half(j, h):
        b = j % _NBUF

        @plsc.parallel_loop(0, _H)
        def _row(r):
            rr = h * _H + r
            for c in range(_BLK):
                w = pe_v[rr, pl.ds(c * 16, 16)]
                lo = lax.bitcast_convert_type(w << 16, jnp.float32)
                hi = lax.bitcast_convert_type(w & jnp.uint32(0xFFFF0000), jnp.float32)
                plsc.addupdate(rows_v.at[b, rr, pl.ds(c * 32, 16)], lo)
                plsc.addupdate(rows_v.at[b, rr, pl.ds(c * 32 + 16, 16)], hi)

    g_descs = [None] * _NCH
    o_descs = [None] * (2 * _NCH)

    for j in range(_NBUF):
        g_descs[j] = start_gather(j)
    pe_desc.wait()
    for j in range(_NCH):
        if j >= 1 and j + _NBUF - 1 < _NCH:
            o_descs[2 * (j - 1)].wait()
            o_descs[2 * (j - 1) + 1].wait()
            g_descs[j + _NBUF - 1] = start_gather(j + _NBUF - 1)
        g_descs[j].wait()
        for h in range(2):
            add_half(j, h)
            row0 = j * _S + s_base + h * _H
            o_descs[2 * j + h] = pltpu.async_copy(
                rows_v.at[j % _NBUF, pl.ds(h * _H, _H)],
                out.at[pl.ds(row0, _H)],
                osem.at[(2 * j + h) % (2 * _NBUF)],
            )
    for j in range(2 * (_NCH - _NBUF), 2 * _NCH):
        o_descs[j].wait()


def kernel(table, x):
    out = _embed_pe(table, jnp.asarray(_PE_U32), x.astype(jnp.int32))
    return out.reshape(_B, _S, _D)


# speedup vs baseline: 1.0508x; 1.0508x over previous
import functools
import math

import ml_dtypes
import numpy as np
import jax
import jax.numpy as jnp
from jax import lax
from jax.experimental import pallas as pl
from jax.experimental.pallas import tpu as pltpu
from jax.experimental.pallas import tpu_sc as plsc

_D = 768
_B = 4
_S = 2048
_NW = 32
_S_PER_W = _S // _NW
_C = 64
_NCH = _B * (_S_PER_W // _C)
_NBUF = 2
_BLK = _D // 32


def _positional_encoding_np(max_len, d_model):
    position = np.arange(max_len, dtype=np.float32)[:, None]
    div_term = np.exp(
        np.arange(0, d_model, 2, dtype=np.float32) * -(math.log(10000.0) / d_model)
    )
    pe = np.zeros((max_len, d_model), dtype=np.float32)
    pe[:, 0::2] = np.sin(position * div_term)
    pe[:, 1::2] = np.cos(position * div_term)
    return pe


_PE = _positional_encoding_np(_S, _D)
_PE_IL = _PE.reshape(_S, _BLK, 2, 16).transpose(0, 1, 3, 2).reshape(_S, _D)
_PE_U32 = (
    _PE_IL.astype(ml_dtypes.bfloat16).view(np.uint32).reshape(_S, _D // 2)
)

_mesh = plsc.VectorSubcoreMesh(core_axis_name="c", subcore_axis_name="s")


@functools.partial(
    pl.kernel,
    mesh=_mesh,
    out_type=jax.ShapeDtypeStruct((_B * _S, _D), jnp.float32),
    scratch_types=[
        pltpu.VMEM((_B, _S_PER_W), jnp.int32),
        pltpu.VMEM((_NBUF, _C, _D), jnp.float32),
        pltpu.VMEM((_S_PER_W, _D // 2), jnp.uint32),
        pltpu.SemaphoreType.DMA((_NBUF,)),
        pltpu.SemaphoreType.DMA,
        pltpu.SemaphoreType.DMA,
        pltpu.SemaphoreType.DMA((_NBUF,)),
    ],
)
def _embed_pe(table, pe, idx, out, idx_v, rows_v, pe_v, gsem, psem, isem, osem):
    wid = lax.axis_index("s") * 2 + lax.axis_index("c")
    s_base = wid * _S_PER_W

    pe_desc = pltpu.async_copy(pe.at[pl.ds(s_base, _S_PER_W)], pe_v, psem)
    idx_descs = [
        pltpu.async_copy(idx.at[b, pl.ds(s_base, _S_PER_W)], idx_v.at[b], isem)
        for b in range(_B)
    ]

    def start_gather(j):
        idx_descs[j].wait()
        return pltpu.async_copy(
            table.at[idx_v.at[j]], rows_v.at[j % _NBUF], gsem.at[j % _NBUF]
        )

    def add_chunk(j):
        b = j % _NBUF

        @plsc.parallel_loop(0, _C)
        def _row(r):
            for c in range(_BLK):
                w = pe_v[r, pl.ds(c * 16, 16)]
                lo = lax.bitcast_convert_type(w << 16, jnp.float32)
                hi = lax.bitcast_convert_type(w & jnp.uint32(0xFFFF0000), jnp.float32)
                plsc.addupdate(rows_v.at[b, r, pl.ds(c * 32, 16)], lo)
                plsc.addupdate(rows_v.at[b, r, pl.ds(c * 32 + 16, 16)], hi)

    g_descs = [None] * _NCH
    o_descs = [None] * _NCH

    for j in range(_NBUF):
        g_descs[j] = start_gather(j)
    pe_desc.wait()
    for j in range(_NCH):
        if j >= 1 and j + _NBUF - 1 < _NCH:
            o_descs[j - 1].wait()
            g_descs[j + _NBUF - 1] = start_gather(j + _NBUF - 1)
        g_descs[j].wait()
        add_chunk(j)
        row0 = j * _S + s_base
        o_descs[j] = pltpu.async_copy(
            rows_v.at[j % _NBUF], out.at[pl.ds(row0, _C)], osem.at[j % _NBUF]
        )
    for j in range(_NCH - _NBUF, _NCH):
        o_descs[j].wait()


def kernel(table, x):
    out = _embed_pe(table, jnp.asarray(_PE_U32), x.astype(jnp.int32))
    return out.reshape(_B, _S, _D)
